# baseline (device time: 350830 ns/iter reference)
import functools
import sys

import jax
import jax.numpy as jnp
from jax import lax
from jax.experimental import pallas as pl
from jax.experimental.pallas import tpu as pltpu

N_DEV = 8

try:
    _ds = jax.devices()
    print(
        "[kernel.py] devices: "
        + "; ".join(
            f"id={d.id} coords={getattr(d, 'coords', None)} "
            f"core={getattr(d, 'core_on_chip', None)}"
            for d in _ds
        ),
        file=sys.stderr,
    )
except Exception as _e:
    print(f"[kernel.py] device probe failed: {_e}", file=sys.stderr)


def kernel(x, w_mat):
    m, _k_shard = x.shape
    n = w_mat.shape[1]
    m_blk = m // N_DEV

    def body(x_ref, w_ref, out_ref, send_buf, recv_bufs, send_sems, recv_sems):
        me = lax.axis_index("i")
        left = lax.rem(me - 1 + N_DEV, N_DEV)
        right = lax.rem(me + 1, N_DEV)

        barrier_sem = pltpu.get_barrier_semaphore()
        for nbr in (left, right):
            pl.semaphore_signal(
                barrier_sem, inc=1, device_id=(nbr,),
                device_id_type=pl.DeviceIdType.MESH,
            )
        pl.semaphore_wait(barrier_sem, 2)

        for t in range(N_DEV):
            b = lax.rem(me - 1 - t + 2 * N_DEV, N_DEV)
            part = jnp.dot(
                x_ref[pl.ds(b * m_blk, m_blk), :],
                w_ref[...],
                preferred_element_type=jnp.float32,
            )
            acc = part if t == 0 else part + recv_bufs[t - 1]
            if t < N_DEV - 1:
                send_buf[...] = acc
                rdma = pltpu.make_async_remote_copy(
                    src_ref=send_buf,
                    dst_ref=recv_bufs.at[t],
                    send_sem=send_sems.at[t],
                    recv_sem=recv_sems.at[t],
                    device_id=(right,),
                    device_id_type=pl.DeviceIdType.MESH,
                )
                rdma.start()
                rdma.wait()
            else:
                out_ref[...] = jnp.maximum(acc, 0.0)

        @functools.partial(
            pl.run_scoped, exit_sem=pltpu.SemaphoreType.REGULAR
        )
        def _(exit_sem):
            for nbr in (left, right):
                pl.semaphore_signal(
                    exit_sem, inc=1, device_id=(nbr,),
                    device_id_type=pl.DeviceIdType.MESH,
                )
            pl.semaphore_wait(exit_sem, 2)

    return pl.pallas_call(
        body,
        out_shape=jax.ShapeDtypeStruct((m_blk, n), jnp.float32),
        in_specs=[
            pl.BlockSpec(memory_space=pltpu.VMEM),
            pl.BlockSpec(memory_space=pltpu.VMEM),
        ],
        out_specs=pl.BlockSpec(memory_space=pltpu.VMEM),
        scratch_shapes=[
            pltpu.VMEM((m_blk, n), jnp.float32),
            pltpu.VMEM((N_DEV - 1, m_blk, n), jnp.float32),
            pltpu.SemaphoreType.DMA((N_DEV - 1,)),
            pltpu.SemaphoreType.DMA((N_DEV - 1,)),
        ],
        compiler_params=pltpu.CompilerParams(collective_id=0),
    )(x, w_mat)


# device time: 200515 ns/iter; 1.7496x vs baseline; 1.7496x over previous
import jax
import jax.numpy as jnp
from jax import lax
from jax.experimental import pallas as pl
from jax.experimental.pallas import tpu as pltpu

N_DEV = 8


def kernel(x, w_mat):
    m, _k_shard = x.shape
    n = w_mat.shape[1]
    m_blk = m // N_DEV
    nh = n // 2

    def body(
        x_ref, w_ref, out_ref,
        send_cw, send_ccw, recv_cw, recv_ccw,
        ssem_cw, ssem_ccw, rsem_cw, rsem_ccw,
    ):
        me = lax.axis_index("i")
        left = lax.rem(me - 1 + N_DEV, N_DEV)
        right = lax.rem(me + 1, N_DEV)

        barrier_sem = pltpu.get_barrier_semaphore()
        for nbr in (left, right):
            pl.semaphore_signal(
                barrier_sem, inc=1, device_id=(nbr,),
                device_id_type=pl.DeviceIdType.MESH,
            )
        pl.semaphore_wait(barrier_sem, 2)

        def dots(t):
            b_cw = lax.rem(me - 1 - t + 2 * N_DEV, N_DEV)
            b_ccw = lax.rem(me + 1 + t, N_DEV)
            p_cw = jnp.dot(
                x_ref[pl.ds(b_cw * m_blk, m_blk), :], w_ref[:, :nh],
                preferred_element_type=jnp.float32,
            )
            p_ccw = jnp.dot(
                x_ref[pl.ds(b_ccw * m_blk, m_blk), :], w_ref[:, nh:],
                preferred_element_type=jnp.float32,
            )
            return p_cw, p_ccw

        def make_rdmas(t, slot):
            r_cw = pltpu.make_async_remote_copy(
                src_ref=send_cw.at[slot],
                dst_ref=recv_cw.at[t],
                send_sem=ssem_cw.at[t],
                recv_sem=rsem_cw.at[t],
                device_id=(right,),
                device_id_type=pl.DeviceIdType.MESH,
            )
            r_ccw = pltpu.make_async_remote_copy(
                src_ref=send_ccw.at[slot],
                dst_ref=recv_ccw.at[t],
                send_sem=ssem_ccw.at[t],
                recv_sem=rsem_ccw.at[t],
                device_id=(left,),
                device_id_type=pl.DeviceIdType.MESH,
            )
            return r_cw, r_ccw

        rdmas = {}
        p_cw, p_ccw = dots(0)
        send_cw[0, :, :] = p_cw
        send_ccw[0, :, :] = p_ccw
        rdmas[0] = make_rdmas(0, 0)
        rdmas[0][0].start()
        rdmas[0][1].start()

        for t in range(1, N_DEV):
            p_cw, p_ccw = dots(t)
            rdmas[t - 1][0].wait_recv()
            rdmas[t - 1][1].wait_recv()
            acc_cw = p_cw + recv_cw[t - 1]
            acc_ccw = p_ccw + recv_ccw[t - 1]
            if t < N_DEV - 1:
                slot = t % 2
                if t >= 2:
                    rdmas[t - 2][0].wait_send()
                    rdmas[t - 2][1].wait_send()
                send_cw[slot, :, :] = acc_cw
                send_ccw[slot, :, :] = acc_ccw
                rdmas[t] = make_rdmas(t, slot)
                rdmas[t][0].start()
                rdmas[t][1].start()
            else:
                out_ref[:, :nh] = jnp.maximum(acc_cw, 0.0)
                out_ref[:, nh:] = jnp.maximum(acc_ccw, 0.0)

        for t in (N_DEV - 3, N_DEV - 2):
            rdmas[t][0].wait_send()
            rdmas[t][1].wait_send()

    return pl.pallas_call(
        body,
        out_shape=jax.ShapeDtypeStruct((m_blk, n), jnp.float32),
        in_specs=[
            pl.BlockSpec(memory_space=pltpu.VMEM),
            pl.BlockSpec(memory_space=pltpu.VMEM),
        ],
        out_specs=pl.BlockSpec(memory_space=pltpu.VMEM),
        scratch_shapes=[
            pltpu.VMEM((2, m_blk, nh), jnp.float32),
            pltpu.VMEM((2, m_blk, nh), jnp.float32),
            pltpu.VMEM((N_DEV - 1, m_blk, nh), jnp.float32),
            pltpu.VMEM((N_DEV - 1, m_blk, nh), jnp.float32),
            pltpu.SemaphoreType.DMA((N_DEV - 1,)),
            pltpu.SemaphoreType.DMA((N_DEV - 1,)),
            pltpu.SemaphoreType.DMA((N_DEV - 1,)),
            pltpu.SemaphoreType.DMA((N_DEV - 1,)),
        ],
        compiler_params=pltpu.CompilerParams(
            collective_id=0,
            vmem_limit_bytes=100 * 1024 * 1024,
        ),
    )(x, w_mat)


# device time: 176649 ns/iter; 1.9860x vs baseline; 1.1351x over previous
import jax
import jax.numpy as jnp
from jax import lax
from jax.experimental import pallas as pl
from jax.experimental.pallas import tpu as pltpu

N_DEV = 8
N_FLOW = 4


def kernel(x, w_mat):
    m, _k_shard = x.shape
    n = w_mat.shape[1]
    m_blk = m // N_DEV
    nc = n // N_FLOW

    def body(x_ref, w_ref, out_ref, send_buf, recv_buf, ssem, rsem):
        me = lax.axis_index("i")
        left = lax.rem(me - 1 + N_DEV, N_DEV)
        right = lax.rem(me + 1, N_DEV)

        barrier_sem = pltpu.get_barrier_semaphore()
        for nbr in (left, right):
            pl.semaphore_signal(
                barrier_sem, inc=1, device_id=(nbr,),
                device_id_type=pl.DeviceIdType.MESH,
            )
        pl.semaphore_wait(barrier_sem, 2)

        def block_idx(t, cw):
            if cw:
                return lax.rem(me - 1 - t + 2 * N_DEV, N_DEV)
            return lax.rem(me + 1 + t, N_DEV)

        def flow_dot(t, f):
            b = block_idx(t, cw=f < 2)
            return jnp.dot(
                x_ref[pl.ds(b * m_blk, m_blk), :],
                w_ref[:, f * nc:(f + 1) * nc],
                preferred_element_type=jnp.float32,
            )

        def make_rdma(t, f, slot):
            return pltpu.make_async_remote_copy(
                src_ref=send_buf.at[f, slot],
                dst_ref=recv_buf.at[f, t],
                send_sem=ssem.at[f, t],
                recv_sem=rsem.at[f, t],
                device_id=(right if f < 2 else left,),
                device_id_type=pl.DeviceIdType.MESH,
            )

        flow_order = (0, 2, 1, 3)

        rdmas = {}
        for f in flow_order:
            p = flow_dot(0, f)
            send_buf[f, 0, :, :] = p
            rdmas[(0, f)] = make_rdma(0, f, 0)
            rdmas[(0, f)].start()

        for t in range(1, N_DEV):
            for f in flow_order:
                p = flow_dot(t, f)
                rdmas[(t - 1, f)].wait_recv()
                acc = p + recv_buf[f, t - 1]
                if t < N_DEV - 1:
                    slot = t % 2
                    if t >= 2:
                        rdmas[(t - 2, f)].wait_send()
                    send_buf[f, slot, :, :] = acc
                    rdmas[(t, f)] = make_rdma(t, f, slot)
                    rdmas[(t, f)].start()
                else:
                    out_ref[:, f * nc:(f + 1) * nc] = jnp.maximum(acc, 0.0)

        for t in (N_DEV - 3, N_DEV - 2):
            for f in flow_order:
                rdmas[(t, f)].wait_send()

    return pl.pallas_call(
        body,
        out_shape=jax.ShapeDtypeStruct((m_blk, n), jnp.float32),
        in_specs=[
            pl.BlockSpec(memory_space=pltpu.VMEM),
            pl.BlockSpec(memory_space=pltpu.VMEM),
        ],
        out_specs=pl.BlockSpec(memory_space=pltpu.VMEM),
        scratch_shapes=[
            pltpu.VMEM((N_FLOW, 2, m_blk, nc), jnp.float32),
            pltpu.VMEM((N_FLOW, N_DEV - 1, m_blk, nc), jnp.float32),
            pltpu.SemaphoreType.DMA((N_FLOW, N_DEV - 1)),
            pltpu.SemaphoreType.DMA((N_FLOW, N_DEV - 1)),
        ],
        compiler_params=pltpu.CompilerParams(
            collective_id=0,
            vmem_limit_bytes=100 * 1024 * 1024,
        ),
    )(x, w_mat)
